# baseline (speedup 1.0000x reference)
import functools

import jax
import jax.numpy as jnp
import numpy as np
from jax import lax
from jax.experimental import pallas as pl
from jax.experimental.pallas import tpu as pltpu
from jax.experimental.pallas import tpu_sc as plsc

N = 10000
NP = 10240
E = 320000
EP = 320256
H = 12
F = 14
HIDP = 192
NW = 32
R = 320
CH = 128
RB = 64

_f32 = jnp.float32


def _edge_body(h_hbm, a_hbm, b_hbm, self_hbm, src_hbm, dst_hbm, bnd_hbm,
               out_hbm, den_hbm,
               bnds, bslab, amax, den, out, sidx, didx, arows,
               hrows, sem_a, sem_h):
    w = lax.axis_index("s") * 2 + lax.axis_index("c")
    base = w * R
    pltpu.sync_copy(bnd_hbm, bnds)
    pltpu.sync_copy(b_hbm.at[pl.ds(base, R)], bslab.at[pl.ds(0, R)])
    pltpu.sync_copy(self_hbm.at[pl.ds(base, R)], den.at[pl.ds(0, R)])
    pltpu.sync_copy(self_hbm.at[pl.ds(base, R)], amax.at[pl.ds(0, R)])
    bv0 = bnds[pl.ds(w, 16)]
    lo = bv0[0]
    hi = bv0[1]
    astart0 = (lo // CH) * CH
    nch = (hi - astart0 + CH - 1) // CH

    def p1_chunk(ci, carry):
        cs = astart0 + ci * CH
        pltpu.sync_copy(src_hbm.at[pl.ds(cs, CH)], sidx)
        pltpu.sync_copy(dst_hbm.at[pl.ds(cs, CH)], didx.at[pl.ds(0, CH)])
        pltpu.async_copy(a_hbm.at[sidx], arows, sem_a).wait()

        def p1_edge(i, c2):
            e = cs + i
            m = jnp.logical_and(e >= lo, e < hi)
            d = jnp.where(m, didx[pl.ds(i, 16)][0] - base, R)
            al = arows[i, pl.ds(0, 16)] + bslab[d]
            al = jnp.maximum(al, 0.2 * al)
            amax[d] = jnp.maximum(amax[d], al)
            return c2

        return lax.fori_loop(0, CH, p1_edge, carry, unroll=8)

    lax.fori_loop(0, nch, p1_chunk, 0)

    def init_blk(rb, carry):
        rbase = rb * RB
        pltpu.sync_copy(h_hbm.at[pl.ds(base + rbase, RB)],
                        hrows.at[pl.ds(0, RB)])

        def init_row(r, c2):
            rr = rbase + r
            ex = jnp.exp(den[rr] - amax[rr])
            den[rr] = ex
            for j in range(H):
                out[rr, pl.ds(16 * j, 16)] = ex[j] * hrows[r, pl.ds(16 * j, 16)]
            return c2

        return lax.fori_loop(0, RB, init_row, carry)

    lax.fori_loop(0, R // RB, init_blk, 0)

    def p2_chunk(ci, carry):
        cs = astart0 + ci * CH
        pltpu.sync_copy(src_hbm.at[pl.ds(cs, CH)], sidx)
        pltpu.sync_copy(dst_hbm.at[pl.ds(cs, CH)], didx.at[pl.ds(0, CH)])
        cpa = pltpu.async_copy(a_hbm.at[sidx], arows, sem_a)
        cph = pltpu.async_copy(h_hbm.at[sidx], hrows, sem_h)
        cpa.wait()
        cph.wait()

        def p2_edge(i, c2):
            e = cs + i
            m = jnp.logical_and(e >= lo, e < hi)
            d = jnp.where(m, didx[pl.ds(i, 16)][0] - base, R)
            al = arows[i, pl.ds(0, 16)] + bslab[d]
            al = jnp.maximum(al, 0.2 * al)
            ex = jnp.exp(al - amax[d])
            den[d] = den[d] + ex
            for j in range(H):
                sl = pl.ds(16 * j, 16)
                out[d, sl] = out[d, sl] + ex[j] * hrows[i, sl]
            return c2

        return lax.fori_loop(0, CH, p2_edge, carry, unroll=4)

    lax.fori_loop(0, nch, p2_chunk, 0)

    pltpu.sync_copy(out.at[pl.ds(0, R)], out_hbm.at[pl.ds(base, R)])
    pltpu.sync_copy(den.at[pl.ds(0, R)], den_hbm.at[pl.ds(base, R)])


@functools.cache
def _get_edge_kernel():
    return pl.kernel(
        _edge_body,
        mesh=plsc.VectorSubcoreMesh(core_axis_name="c", subcore_axis_name="s"),
        compiler_params=pltpu.CompilerParams(use_tc_tiling_on_sc=False),
        out_type=[
            jax.ShapeDtypeStruct((NP, HIDP), _f32),
            jax.ShapeDtypeStruct((NP, 16), _f32),
        ],
        scratch_types=[
            pltpu.VMEM((48,), jnp.int32),
            pltpu.VMEM((R + 8, 16), _f32),
            pltpu.VMEM((R + 8, 16), _f32),
            pltpu.VMEM((R + 8, 16), _f32),
            pltpu.VMEM((R + 8, HIDP), _f32),
            pltpu.VMEM((CH,), jnp.int32),
            pltpu.VMEM((CH + 16,), jnp.int32),
            pltpu.VMEM((CH, 128), _f32),
            pltpu.VMEM((CH, 256), _f32),
            pltpu.SemaphoreType.DMA,
            pltpu.SemaphoreType.DMA,
        ],
    )


def _dense_body(x_ref, g_ref, bb_ref, w_ref, was_ref, wad_ref,
                h_ref, a_ref, b_ref, self_ref):
    xb = x_ref[...] * g_ref[...] + bb_ref[...]
    h_ref[...] = jnp.dot(xb, w_ref[...], preferred_element_type=_f32, precision=jax.lax.Precision.HIGHEST)
    av = jnp.dot(xb, was_ref[...], preferred_element_type=_f32, precision=jax.lax.Precision.HIGHEST)
    bv = jnp.dot(xb, wad_ref[...], preferred_element_type=_f32, precision=jax.lax.Precision.HIGHEST)
    a_ref[...] = av
    b_ref[...] = bv
    sv = av[:, :16] + bv
    self_ref[...] = jnp.maximum(sv, 0.2 * sv)


def _dense_call(xp, g, bb, wp, was, wad):
    k = xp.shape[1]
    bs = 512
    grid = NP // bs
    return pl.pallas_call(
        _dense_body,
        grid=(grid,),
        in_specs=[
            pl.BlockSpec((bs, k), lambda i: (i, 0)),
            pl.BlockSpec((1, k), lambda i: (0, 0)),
            pl.BlockSpec((1, k), lambda i: (0, 0)),
            pl.BlockSpec((k, 256), lambda i: (0, 0)),
            pl.BlockSpec((k, 128), lambda i: (0, 0)),
            pl.BlockSpec((k, 16), lambda i: (0, 0)),
        ],
        out_specs=[
            pl.BlockSpec((bs, 256), lambda i: (i, 0)),
            pl.BlockSpec((bs, 128), lambda i: (i, 0)),
            pl.BlockSpec((bs, 16), lambda i: (i, 0)),
            pl.BlockSpec((bs, 16), lambda i: (i, 0)),
        ],
        out_shape=[
            jax.ShapeDtypeStruct((NP, 256), _f32),
            jax.ShapeDtypeStruct((NP, 128), _f32),
            jax.ShapeDtypeStruct((NP, 16), _f32),
            jax.ShapeDtypeStruct((NP, 16), _f32),
        ],
    )(xp, g, bb, wp, was, wad)


def _finalize_body(out_ref, den_ref, bias_ref, x_ref):
    q = out_ref[...]
    den = den_ref[...] + 1e-16
    parts = [q[:, 16 * j:16 * (j + 1)] / den[:, j:j + 1] for j in range(H)]
    qn = jnp.concatenate(parts, axis=1)
    v = qn + bias_ref[...]
    x_ref[...] = jnp.where(v > 0, v, (jnp.exp(v) - 1.0))


def _finalize_call(out_sc, den, bias_p):
    bs = 512
    return pl.pallas_call(
        _finalize_body,
        grid=(NP // bs,),
        in_specs=[
            pl.BlockSpec((bs, HIDP), lambda i: (i, 0)),
            pl.BlockSpec((bs, 16), lambda i: (i, 0)),
            pl.BlockSpec((1, HIDP), lambda i: (0, 0)),
        ],
        out_specs=pl.BlockSpec((bs, HIDP), lambda i: (i, 0)),
        out_shape=jax.ShapeDtypeStruct((NP, HIDP), _f32),
    )(out_sc, den, bias_p)


def _lastpool_body(out_ref, den_ref, bias_ref, s_ref):
    q = out_ref[...]
    den = den_ref[...] + 1e-16
    acc = q[:, 0:16] / den[:, 0:1]
    for j in range(1, H):
        acc = acc + q[:, 16 * j:16 * (j + 1)] / den[:, j:j + 1]
    y = acc * (1.0 / H) + bias_ref[...]
    bs = y.shape[0]
    rows = pl.program_id(0) * bs + lax.broadcasted_iota(jnp.int32, (bs, 1), 0)
    y = jnp.where(rows < N, y, 0.0)
    s_ref[...] = jnp.sum(y, axis=0, keepdims=True)[None]


def _lastpool_call(out_sc, den, bias16):
    bs = 512
    return pl.pallas_call(
        _lastpool_body,
        grid=(NP // bs,),
        in_specs=[
            pl.BlockSpec((bs, HIDP), lambda i: (i, 0)),
            pl.BlockSpec((bs, 16), lambda i: (i, 0)),
            pl.BlockSpec((1, 16), lambda i: (0, 0)),
        ],
        out_specs=pl.BlockSpec((1, 1, 16), lambda i: (i, 0, 0)),
        out_shape=jax.ShapeDtypeStruct((NP // bs, 1, 16), _f32),
    )(out_sc, den, bias16)


def _head_body(cm_ref, gf_ref, w1_ref, b1_ref, w2_ref, b2_ref,
               w0a_ref, w0b_ref, b0_ref, wm_refs_and_out):
    wm = wm_refs_and_out[:-2]
    wo_ref, bo_ref = wm_refs_and_out[-2][0], wm_refs_and_out[-2][1]
    o_ref = wm_refs_and_out[-1]
    p = jnp.dot(cm_ref[...], w1_ref[...], preferred_element_type=_f32, precision=jax.lax.Precision.HIGHEST) + b1_ref[...]
    p = jnp.dot(p, w2_ref[...], preferred_element_type=_f32, precision=jax.lax.Precision.HIGHEST) + b2_ref[...]
    t = (jnp.dot(p, w0a_ref[...], preferred_element_type=_f32, precision=jax.lax.Precision.HIGHEST)
         + jnp.dot(gf_ref[...], w0b_ref[...], preferred_element_type=_f32, precision=jax.lax.Precision.HIGHEST)
         + b0_ref[...])
    t = jnp.where(t > 0, t, (jnp.exp(t) - 1.0))
    for wr, br in wm:
        t = jnp.dot(t, wr[...], preferred_element_type=_f32, precision=jax.lax.Precision.HIGHEST) + br[...]
        t = jnp.where(t > 0, t, (jnp.exp(t) - 1.0))
        t = jnp.where(t > 0, t, (jnp.exp(t) - 1.0))
    o = jnp.dot(t, wo_ref[...], preferred_element_type=_f32, precision=jax.lax.Precision.HIGHEST) + bo_ref[...]
    o_ref[...] = jax.nn.sigmoid(o)


def _head_flat_body(cm_ref, gf_ref, w1_ref, b1_ref, w2_ref, b2_ref,
                    w0a_ref, w0b_ref, b0_ref,
                    m1w, m1b, m2w, m2b, m3w, m3b, m4w, m4b, m5w, m5b, m6w, m6b,
                    wo_ref, bo_ref, o_ref):
    wm = [(m1w, m1b), (m2w, m2b), (m3w, m3b), (m4w, m4b), (m5w, m5b), (m6w, m6b)]
    _head_body(cm_ref, gf_ref, w1_ref, b1_ref, w2_ref, b2_ref,
               w0a_ref, w0b_ref, b0_ref, wm + [(wo_ref, bo_ref), o_ref])


def _head_call(cm, gfp, w1f, b1f, w2, b2, w0a, w0b, b0, mids, wo, bo):
    args = [cm, gfp, w1f, b1f, w2, b2, w0a, w0b, b0]
    for wr, br in mids:
        args += [wr, br]
    args += [wo, bo]
    in_specs = [pl.BlockSpec(a.shape, lambda i: tuple(0 for _ in a.shape))
                for a in args]
    return pl.pallas_call(
        _head_flat_body,
        grid=(1,),
        in_specs=in_specs,
        out_specs=pl.BlockSpec((1, 1), lambda i: (0, 0)),
        out_shape=jax.ShapeDtypeStruct((1, 1), _f32),
    )(*args)


_IDX168 = (np.arange(H * F) // F) * 16 + (np.arange(H * F) % F)
_BN_INV = float(1.0 / np.sqrt(1.0 + 1e-5))


def _pad_cols(w):
    return jnp.zeros((w.shape[0], 256), _f32).at[:, _IDX168].set(w)


def _pad_rows(w):
    return jnp.zeros((HIDP, w.shape[1]), _f32).at[_IDX168].set(w)


def _fold_att(w, avec, width=16):
    wr = w.reshape(w.shape[0], H, F)
    a = jnp.einsum('khf,hf->kh', wr, avec)
    return jnp.pad(a, ((0, 0), (0, width - H)))


def kernel(x, edge_index, edge_attr, batch, gf1, params):
    src = edge_index[0].astype(jnp.int32)
    dst = edge_index[1].astype(jnp.int32)
    order = jnp.argsort(dst)
    srcs = jnp.concatenate([src[order], jnp.zeros((EP - E,), jnp.int32)])
    dsts_real = dst[order]
    dsts = jnp.concatenate([dsts_real, jnp.zeros((EP - E,), jnp.int32)])
    bnd = jnp.searchsorted(
        dsts_real, jnp.arange(NW + 1, dtype=jnp.int32) * R).astype(jnp.int32)
    bnd = jnp.concatenate([bnd, jnp.zeros((15,), jnp.int32)])

    xp = jnp.pad(x.T, ((0, NP - N), (0, 0)))

    for i in range(8):
        w = params['gat%d_W' % i]
        k = w.shape[0]
        if i == 0:
            wp = _pad_cols(w)
            was = _fold_att(w, params['gat%d_asrc' % i], 128)
            wad = _fold_att(w, params['gat%d_adst' % i])
            g = (params['bn%d_g' % i] * _BN_INV).reshape(1, k)
            bb = params['bn%d_b' % i].reshape(1, k)
        else:
            wp = _pad_rows(_pad_cols(w))
            was = _pad_rows(_fold_att(w, params['gat%d_asrc' % i], 128))
            wad = _pad_rows(_fold_att(w, params['gat%d_adst' % i]))
            g = jnp.zeros((1, HIDP), _f32).at[0, _IDX168].set(
                params['bn%d_g' % i] * _BN_INV)
            bb = jnp.zeros((1, HIDP), _f32).at[0, _IDX168].set(
                params['bn%d_b' % i])

        h2, av, bv, sv = _dense_call(xp, g, bb, wp, was, wad)
        out2, den = _get_edge_kernel()(h2, av, bv, sv, srcs, dsts, bnd)

        if i < 7:
            bias_p = jnp.zeros((1, HIDP), _f32).at[0, _IDX168].set(
                params['gat%d_b' % i])
            xp = _finalize_call(out2, den, bias_p)
        else:
            bias16 = jnp.pad(params['gat7_b'], (0, 2)).reshape(1, 16)
            blocksums = _lastpool_call(out2, den, bias16)

    cm = (blocksums.sum(axis=(0, 1))[None, :] / N).astype(_f32)

    s14 = params['bnpool_g'] * _BN_INV
    w1f = jnp.pad(s14[:, None] * params['ugp1_W'], ((0, 2), (0, 0)))
    b1f = (params['bnpool_b'] @ params['ugp1_W']
           + params['ugp1_b']).reshape(1, -1)
    w2 = params['ugp2_W']
    b2 = params['ugp2_b'].reshape(1, -1)
    w0 = params['dnn0_W']
    w0a = w0[:50]
    w0b = jnp.pad(w0[50:], ((0, 3), (0, 0)))
    b0 = params['dnn0_b'].reshape(1, -1)
    gfp = jnp.pad(gf1, (0, 3)).reshape(1, 48)
    mids = [(params['dnn%d_W' % i], params['dnn%d_b' % i].reshape(1, -1))
            for i in range(1, 7)]
    wo = params['dnnout_W']
    bo = params['dnnout_b'].reshape(1, 1)

    return _head_call(cm, gfp, w1f, b1f, w2, b2, w0a, w0b, b0, mids, wo, bo)


# speedup vs baseline: 1.2010x; 1.2010x over previous
import functools

import jax
import jax.numpy as jnp
import numpy as np
from jax import lax
from jax.experimental import pallas as pl
from jax.experimental.pallas import tpu as pltpu
from jax.experimental.pallas import tpu_sc as plsc

N = 10000
NP = 10240
E = 320000
EP = 320256
H = 12
F = 14
HIDP = 192
NW = 32
R = 320
CH = 64
RB = 64

_f32 = jnp.float32
_HIGH = jax.lax.Precision.HIGHEST


def _edge_body(h_hbm, a_hbm, b_hbm, self_hbm, src_hbm, dst_hbm, bnd_hbm,
               out_hbm, den_hbm,
               bnds, bslab, amax, den, out,
               sidx0, sidx1, didx0, didx1, arows0, arows1, hrows0, hrows1,
               semi0, semi1, sema0, sema1, semh0, semh1):
    w = lax.axis_index("s") * 2 + lax.axis_index("c")
    base = w * R
    pltpu.sync_copy(bnd_hbm, bnds)
    pltpu.sync_copy(b_hbm.at[pl.ds(base, R)], bslab.at[pl.ds(0, R)])
    pltpu.sync_copy(self_hbm.at[pl.ds(base, R)], den.at[pl.ds(0, R)])
    pltpu.sync_copy(self_hbm.at[pl.ds(base, R)], amax.at[pl.ds(0, R)])
    bv0 = bnds[pl.ds(w, 16)]
    lo = bv0[0]
    hi = bv0[1]
    astart0 = (lo // CH) * CH
    nch = (hi - astart0 + CH - 1) // CH

    sidx = (sidx0, sidx1)
    didx = (didx0, didx1)
    arows = (arows0, arows1)
    hrows = (hrows0, hrows1)
    semi = (semi0, semi1)
    sema = (sema0, sema1)
    semh = (semh0, semh1)

    def issue_idx(c, buf):
        @pl.when(c < nch)
        def _():
            cs = astart0 + c * CH
            pltpu.async_copy(src_hbm.at[pl.ds(cs, CH)], sidx[buf], semi[buf])
            pltpu.async_copy(dst_hbm.at[pl.ds(cs, CH)],
                             didx[buf].at[pl.ds(0, CH)], semi[buf])

    def wait_idx(buf):
        pltpu.make_async_copy(src_hbm.at[pl.ds(0, CH)], sidx[buf],
                              semi[buf]).wait()
        pltpu.make_async_copy(dst_hbm.at[pl.ds(0, CH)],
                              didx[buf].at[pl.ds(0, CH)], semi[buf]).wait()

    def issue_gather(buf, with_h):
        pltpu.async_copy(a_hbm.at[sidx[buf]], arows[buf], sema[buf])
        if with_h:
            pltpu.async_copy(h_hbm.at[sidx[buf]], hrows[buf], semh[buf])

    def wait_gather(buf, with_h):
        pltpu.make_async_copy(a_hbm.at[sidx[buf]], arows[buf],
                              sema[buf]).wait()
        if with_h:
            pltpu.make_async_copy(h_hbm.at[sidx[buf]], hrows[buf],
                                  semh[buf]).wait()

    def run_pass(edge_fn, with_h, unroll):
        issue_idx(jnp.int32(0), 0)
        issue_idx(jnp.int32(1), 1)

        @pl.when(jnp.int32(0) < nch)
        def _():
            wait_idx(0)
            issue_gather(0, with_h)

        def outer(cb, carry):
            for b in (0, 1):
                c = cb * 2 + b
                nbuf = b ^ 1

                @pl.when(c + 1 < nch)
                def _():
                    wait_idx(nbuf)
                    issue_gather(nbuf, with_h)

                @pl.when(c < nch)
                def _():
                    wait_gather(b, with_h)
                    cs = astart0 + c * CH

                    def inner(i, c2):
                        e = cs + i
                        m = jnp.logical_and(e >= lo, e < hi)
                        d = jnp.where(m, didx[b][pl.ds(i, 16)][0] - base, R)
                        edge_fn(i, d, b)
                        return c2

                    lax.fori_loop(0, CH, inner, 0, unroll=unroll)

                issue_idx(c + 2, b)
            return carry

        lax.fori_loop(0, (nch + 1) // 2, outer, 0)

    def p1_edge(i, d, b):
        al = arows[b][i, pl.ds(0, 16)] + bslab[d]
        al = jnp.maximum(al, 0.2 * al)
        amax[d] = jnp.maximum(amax[d], al)

    run_pass(p1_edge, with_h=False, unroll=1)

    def init_blk(rb, carry):
        rbase = rb * RB
        pltpu.sync_copy(h_hbm.at[pl.ds(base + rbase, RB)],
                        hrows0.at[pl.ds(0, RB)])

        def init_row(r, c2):
            rr = rbase + r
            ex = jnp.exp(den[rr] - amax[rr])
            den[rr] = ex
            for j in range(H):
                sl = pl.ds(16 * j, 16)
                out[rr, sl] = ex[j] * hrows0[r, sl]
            return c2

        return lax.fori_loop(0, RB, init_row, carry)

    lax.fori_loop(0, R // RB, init_blk, 0)

    def p2_edge(i, d, b):
        al = arows[b][i, pl.ds(0, 16)] + bslab[d]
        al = jnp.maximum(al, 0.2 * al)
        ex = jnp.exp(al - amax[d])
        den[d] = den[d] + ex
        for j in range(H):
            sl = pl.ds(16 * j, 16)
            out[d, sl] = out[d, sl] + ex[j] * hrows[b][i, sl]

    run_pass(p2_edge, with_h=True, unroll=1)

    pltpu.sync_copy(out.at[pl.ds(0, R)], out_hbm.at[pl.ds(base, R)])
    pltpu.sync_copy(den.at[pl.ds(0, R)], den_hbm.at[pl.ds(base, R)])


@functools.cache
def _get_edge_kernel():
    return pl.kernel(
        _edge_body,
        mesh=plsc.VectorSubcoreMesh(core_axis_name="c", subcore_axis_name="s"),
        compiler_params=pltpu.CompilerParams(use_tc_tiling_on_sc=False),
        out_type=[
            jax.ShapeDtypeStruct((NP, HIDP), _f32),
            jax.ShapeDtypeStruct((NP, 16), _f32),
        ],
        scratch_types=[
            pltpu.VMEM((48,), jnp.int32),
            pltpu.VMEM((R + 8, 16), _f32),
            pltpu.VMEM((R + 8, 16), _f32),
            pltpu.VMEM((R + 8, 16), _f32),
            pltpu.VMEM((R + 8, HIDP), _f32),
            pltpu.VMEM((CH,), jnp.int32),
            pltpu.VMEM((CH,), jnp.int32),
            pltpu.VMEM((CH + 16,), jnp.int32),
            pltpu.VMEM((CH + 16,), jnp.int32),
            pltpu.VMEM((CH, 128), _f32),
            pltpu.VMEM((CH, 128), _f32),
            pltpu.VMEM((CH, 256), _f32),
            pltpu.VMEM((CH, 256), _f32),
            pltpu.SemaphoreType.DMA,
            pltpu.SemaphoreType.DMA,
            pltpu.SemaphoreType.DMA,
            pltpu.SemaphoreType.DMA,
            pltpu.SemaphoreType.DMA,
            pltpu.SemaphoreType.DMA,
        ],
    )


def _dense_body(x_ref, g_ref, bb_ref, w_ref, was_ref, wad_ref,
                h_ref, a_ref, b_ref, self_ref):
    xb = x_ref[...] * g_ref[...] + bb_ref[...]
    h_ref[...] = jnp.dot(xb, w_ref[...], preferred_element_type=_f32,
                         precision=_HIGH)
    av = jnp.dot(xb, was_ref[...], preferred_element_type=_f32,
                 precision=_HIGH)
    bv = jnp.dot(xb, wad_ref[...], preferred_element_type=_f32,
                 precision=_HIGH)
    a_ref[...] = av
    b_ref[...] = bv
    sv = av[:, :16] + bv
    self_ref[...] = jnp.maximum(sv, 0.2 * sv)


def _dense_call(xp, g, bb, wp, was, wad):
    k = xp.shape[1]
    bs = 512
    grid = NP // bs
    return pl.pallas_call(
        _dense_body,
        grid=(grid,),
        in_specs=[
            pl.BlockSpec((bs, k), lambda i: (i, 0)),
            pl.BlockSpec((1, k), lambda i: (0, 0)),
            pl.BlockSpec((1, k), lambda i: (0, 0)),
            pl.BlockSpec((k, 256), lambda i: (0, 0)),
            pl.BlockSpec((k, 128), lambda i: (0, 0)),
            pl.BlockSpec((k, 16), lambda i: (0, 0)),
        ],
        out_specs=[
            pl.BlockSpec((bs, 256), lambda i: (i, 0)),
            pl.BlockSpec((bs, 128), lambda i: (i, 0)),
            pl.BlockSpec((bs, 16), lambda i: (i, 0)),
            pl.BlockSpec((bs, 16), lambda i: (i, 0)),
        ],
        out_shape=[
            jax.ShapeDtypeStruct((NP, 256), _f32),
            jax.ShapeDtypeStruct((NP, 128), _f32),
            jax.ShapeDtypeStruct((NP, 16), _f32),
            jax.ShapeDtypeStruct((NP, 16), _f32),
        ],
    )(xp, g, bb, wp, was, wad)


def _finalize_body(out_ref, den_ref, bias_ref, x_ref):
    q = out_ref[...]
    den = den_ref[...] + 1e-16
    parts = [q[:, 16 * j:16 * (j + 1)] / den[:, j:j + 1] for j in range(H)]
    qn = jnp.concatenate(parts, axis=1)
    v = qn + bias_ref[...]
    x_ref[...] = jnp.where(v > 0, v, (jnp.exp(v) - 1.0))


def _finalize_call(out_sc, den, bias_p):
    bs = 512
    return pl.pallas_call(
        _finalize_body,
        grid=(NP // bs,),
        in_specs=[
            pl.BlockSpec((bs, HIDP), lambda i: (i, 0)),
            pl.BlockSpec((bs, 16), lambda i: (i, 0)),
            pl.BlockSpec((1, HIDP), lambda i: (0, 0)),
        ],
        out_specs=pl.BlockSpec((bs, HIDP), lambda i: (i, 0)),
        out_shape=jax.ShapeDtypeStruct((NP, HIDP), _f32),
    )(out_sc, den, bias_p)


def _lastpool_body(out_ref, den_ref, bias_ref, s_ref):
    q = out_ref[...]
    den = den_ref[...] + 1e-16
    acc = q[:, 0:16] / den[:, 0:1]
    for j in range(1, H):
        acc = acc + q[:, 16 * j:16 * (j + 1)] / den[:, j:j + 1]
    y = acc * (1.0 / H) + bias_ref[...]
    bs = y.shape[0]
    rows = pl.program_id(0) * bs + lax.broadcasted_iota(jnp.int32, (bs, 1), 0)
    y = jnp.where(rows < N, y, 0.0)
    s_ref[...] = jnp.sum(y, axis=0, keepdims=True)[None]


def _lastpool_call(out_sc, den, bias16):
    bs = 512
    return pl.pallas_call(
        _lastpool_body,
        grid=(NP // bs,),
        in_specs=[
            pl.BlockSpec((bs, HIDP), lambda i: (i, 0)),
            pl.BlockSpec((bs, 16), lambda i: (i, 0)),
            pl.BlockSpec((1, 16), lambda i: (0, 0)),
        ],
        out_specs=pl.BlockSpec((1, 1, 16), lambda i: (i, 0, 0)),
        out_shape=jax.ShapeDtypeStruct((NP // bs, 1, 16), _f32),
    )(out_sc, den, bias16)


def _head_body(cm_ref, gf_ref, w1_ref, b1_ref, w2_ref, b2_ref,
               w0a_ref, w0b_ref, b0_ref, wm_refs_and_out):
    wm = wm_refs_and_out[:-2]
    wo_ref, bo_ref = wm_refs_and_out[-2][0], wm_refs_and_out[-2][1]
    o_ref = wm_refs_and_out[-1]
    p = jnp.dot(cm_ref[...], w1_ref[...], preferred_element_type=_f32,
                precision=_HIGH) + b1_ref[...]
    p = jnp.dot(p, w2_ref[...], preferred_element_type=_f32,
                precision=_HIGH) + b2_ref[...]
    t = (jnp.dot(p, w0a_ref[...], preferred_element_type=_f32,
                 precision=_HIGH)
         + jnp.dot(gf_ref[...], w0b_ref[...], preferred_element_type=_f32,
                   precision=_HIGH)
         + b0_ref[...])
    t = jnp.where(t > 0, t, (jnp.exp(t) - 1.0))
    for wr, br in wm:
        t = jnp.dot(t, wr[...], preferred_element_type=_f32,
                    precision=_HIGH) + br[...]
        t = jnp.where(t > 0, t, (jnp.exp(t) - 1.0))
        t = jnp.where(t > 0, t, (jnp.exp(t) - 1.0))
    o = jnp.dot(t, wo_ref[...], preferred_element_type=_f32,
                precision=_HIGH) + bo_ref[...]
    o_ref[...] = jax.nn.sigmoid(o)


def _head_flat_body(cm_ref, gf_ref, w1_ref, b1_ref, w2_ref, b2_ref,
                    w0a_ref, w0b_ref, b0_ref,
                    m1w, m1b, m2w, m2b, m3w, m3b, m4w, m4b, m5w, m5b, m6w, m6b,
                    wo_ref, bo_ref, o_ref):
    wm = [(m1w, m1b), (m2w, m2b), (m3w, m3b), (m4w, m4b), (m5w, m5b), (m6w, m6b)]
    _head_body(cm_ref, gf_ref, w1_ref, b1_ref, w2_ref, b2_ref,
               w0a_ref, w0b_ref, b0_ref, wm + [(wo_ref, bo_ref), o_ref])


def _head_call(cm, gfp, w1f, b1f, w2, b2, w0a, w0b, b0, mids, wo, bo):
    args = [cm, gfp, w1f, b1f, w2, b2, w0a, w0b, b0]
    for wr, br in mids:
        args += [wr, br]
    args += [wo, bo]
    in_specs = [pl.BlockSpec(a.shape, lambda i: tuple(0 for _ in a.shape))
                for a in args]
    return pl.pallas_call(
        _head_flat_body,
        grid=(1,),
        in_specs=in_specs,
        out_specs=pl.BlockSpec((1, 1), lambda i: (0, 0)),
        out_shape=jax.ShapeDtypeStruct((1, 1), _f32),
    )(*args)


_IDX168 = (np.arange(H * F) // F) * 16 + (np.arange(H * F) % F)
_BN_INV = float(1.0 / np.sqrt(1.0 + 1e-5))


def _pad_cols(w):
    return jnp.zeros((w.shape[0], 256), _f32).at[:, _IDX168].set(w)


def _pad_rows(w):
    return jnp.zeros((HIDP, w.shape[1]), _f32).at[_IDX168].set(w)


def _fold_att(w, avec, width=16):
    wr = w.reshape(w.shape[0], H, F)
    a = jnp.einsum('khf,hf->kh', wr, avec)
    return jnp.pad(a, ((0, 0), (0, width - H)))


def kernel(x, edge_index, edge_attr, batch, gf1, params):
    src = edge_index[0].astype(jnp.int32)
    dst = edge_index[1].astype(jnp.int32)
    order = jnp.argsort(dst)
    srcs = jnp.concatenate([src[order], jnp.zeros((EP - E,), jnp.int32)])
    dsts_real = dst[order]
    dsts = jnp.concatenate([dsts_real, jnp.zeros((EP - E,), jnp.int32)])
    bnd = jnp.searchsorted(
        dsts_real, jnp.arange(NW + 1, dtype=jnp.int32) * R).astype(jnp.int32)
    bnd = jnp.concatenate([bnd, jnp.zeros((15,), jnp.int32)])

    xp = jnp.pad(x.T, ((0, NP - N), (0, 0)))

    for i in range(8):
        w = params['gat%d_W' % i]
        k = w.shape[0]
        if i == 0:
            wp = _pad_cols(w)
            was = _fold_att(w, params['gat%d_asrc' % i], 128)
            wad = _fold_att(w, params['gat%d_adst' % i])
            g = (params['bn%d_g' % i] * _BN_INV).reshape(1, k)
            bb = params['bn%d_b' % i].reshape(1, k)
        else:
            wp = _pad_rows(_pad_cols(w))
            was = _pad_rows(_fold_att(w, params['gat%d_asrc' % i], 128))
            wad = _pad_rows(_fold_att(w, params['gat%d_adst' % i]))
            g = jnp.zeros((1, HIDP), _f32).at[0, _IDX168].set(
                params['bn%d_g' % i] * _BN_INV)
            bb = jnp.zeros((1, HIDP), _f32).at[0, _IDX168].set(
                params['bn%d_b' % i])

        h2, av, bv, sv = _dense_call(xp, g, bb, wp, was, wad)
        out2, den = _get_edge_kernel()(h2, av, bv, sv, srcs, dsts, bnd)

        if i < 7:
            bias_p = jnp.zeros((1, HIDP), _f32).at[0, _IDX168].set(
                params['gat%d_b' % i])
            xp = _finalize_call(out2, den, bias_p)
        else:
            bias16 = jnp.pad(params['gat7_b'], (0, 2)).reshape(1, 16)
            blocksums = _lastpool_call(out2, den, bias16)

    cm = (blocksums.sum(axis=(0, 1))[None, :] / N).astype(_f32)

    s14 = params['bnpool_g'] * _BN_INV
    w1f = jnp.pad(s14[:, None] * params['ugp1_W'], ((0, 2), (0, 0)))
    b1f = (params['bnpool_b'] @ params['ugp1_W']
           + params['ugp1_b']).reshape(1, -1)
    w2 = params['ugp2_W']
    b2 = params['ugp2_b'].reshape(1, -1)
    w0 = params['dnn0_W']
    w0a = w0[:50]
    w0b = jnp.pad(w0[50:], ((0, 3), (0, 0)))
    b0 = params['dnn0_b'].reshape(1, -1)
    gfp = jnp.pad(gf1, (0, 3)).reshape(1, 48)
    mids = [(params['dnn%d_W' % i], params['dnn%d_b' % i].reshape(1, -1))
            for i in range(1, 7)]
    wo = params['dnnout_W']
    bo = params['dnnout_b'].reshape(1, 1)

    return _head_call(cm, gfp, w1f, b1f, w2, b2, w0a, w0b, b0, mids, wo, bo)
